# baseline (speedup 1.0000x reference)
import functools

import jax
import jax.numpy as jnp
from jax import lax
from jax.experimental import pallas as pl
from jax.experimental.pallas import tpu as pltpu
import jax.experimental.pallas.tpu_sc as plsc

_E = 8
_TOPK = 2
_N = 8192
_D = 1024
_DFF = 4096
_A = _N * _TOPK

_TM = 512
_TKD = 2048
_NT = _A // _TM
_G = _NT + _E - 1
_KD = _DFF // _TKD

_NC = 2
_NS = 16
_NW = _NC * _NS


def _dest_body(ei_ref, cnt_ref, dest_ref, meta_ref):
    ei = ei_ref[...]

    r = lax.broadcasted_iota(jnp.int32, (128, 128), 0)
    c = lax.broadcasted_iota(jnp.int32, (128, 128), 1)
    tlt = (r < c).astype(jnp.float32)
    slo = (c < r).astype(jnp.float32)

    dest = jnp.zeros((128, 128), jnp.float32)
    offs = [jnp.int32(0)]
    for e in range(_E):
        plane = (ei == e).astype(jnp.float32)
        col_ex = jnp.dot(plane, tlt, preferred_element_type=jnp.float32)
        rowsum = jnp.sum(plane, axis=1, keepdims=True)
        row_ex = jnp.dot(slo, rowsum, preferred_element_type=jnp.float32)
        rank = col_ex + row_ex
        dest = dest + plane * (offs[e].astype(jnp.float32) + rank)
        offs.append(offs[e] + cnt_ref[0, e])
    dest_ref[...] = dest.astype(jnp.int32)

    start_t, units = [], [jnp.int32(0)]
    for e in range(_E):
        start_t.append(offs[e] // _TM)
        cnt_e = offs[e + 1] - offs[e]
        end_t = jnp.where(cnt_e > 0, (offs[e + 1] - 1) // _TM, start_t[e] - 1)
        units.append(units[e] + jnp.maximum(end_t - start_t[e] + 1, 0))
    g = lax.broadcasted_iota(jnp.int32, (1, 128), 1)
    e_id = jnp.zeros((1, 128), jnp.int32)
    for e in range(_E - 1):
        e_id = e_id + (g >= units[e + 1]).astype(jnp.int32)
    e_id = jnp.minimum(e_id, _E - 1)
    tile = jnp.zeros((1, 128), jnp.int32)
    off_e = jnp.zeros((1, 128), jnp.int32)
    end_e = jnp.zeros((1, 128), jnp.int32)
    for e in range(_E):
        sel = (e_id == e).astype(jnp.int32)
        tile = tile + sel * (start_t[e] + g - units[e])
        off_e = off_e + sel * offs[e]
        end_e = end_e + sel * offs[e + 1]
    valid = (g < units[_E]).astype(jnp.int32)
    tile = jnp.where(valid == 1, tile, _NT - 1)
    lo = valid * jnp.maximum(off_e, tile * _TM)
    hi = valid * jnp.minimum(end_e, (tile + 1) * _TM)
    e_row = valid * e_id
    z = jnp.zeros((1, 128), jnp.int32)
    meta_ref[...] = jnp.concatenate(
        [tile, e_row, lo, hi, z, z, z, z], axis=0)


def _dest(ei2d, cnt):
    return pl.pallas_call(
        _dest_body,
        in_specs=[
            pl.BlockSpec((128, 128), lambda: (0, 0)),
            pl.BlockSpec(memory_space=pltpu.SMEM),
        ],
        out_shape=(jax.ShapeDtypeStruct((128, 128), jnp.int32),
                   jax.ShapeDtypeStruct((8, 128), jnp.int32)),
    )(ei2d, cnt)


def _dispatch(x, dest01):
    tpw = _N // _NW
    ch = 64
    nch = tpw // ch
    mesh = plsc.VectorSubcoreMesh(core_axis_name="c", subcore_axis_name="s")

    @functools.partial(
        pl.kernel,
        out_type=jax.ShapeDtypeStruct((_A, _D), jnp.float32),
        mesh=mesh,
        scratch_types=[
            pltpu.VMEM((ch, _D), jnp.float32),
            pltpu.VMEM((ch,), jnp.int32),
            pltpu.VMEM((ch,), jnp.int32),
            pltpu.SemaphoreType.DMA,
            pltpu.SemaphoreType.DMA,
        ],
    )
    def k(x_hbm, d_hbm, xs_hbm, rows_v, i0_v, i1_v, s0, s1):
        wid = lax.axis_index("s") * _NC + lax.axis_index("c")
        base = wid * tpw
        for cnk in range(nch):
            t0 = base + cnk * ch
            pltpu.sync_copy(x_hbm.at[pl.ds(t0, ch)], rows_v)
            pltpu.sync_copy(d_hbm.at[0, pl.ds(t0, ch)], i0_v)
            pltpu.sync_copy(d_hbm.at[1, pl.ds(t0, ch)], i1_v)
            c0 = pltpu.async_copy(rows_v, xs_hbm.at[i0_v], s0)
            c1 = pltpu.async_copy(rows_v, xs_hbm.at[i1_v], s1)
            c0.wait()
            c1.wait()

    return k(x, dest01)


def _gmm_body(meta_ref, xs_ref, w1_ref, w2_ref, ys_ref):
    g = pl.program_id(0)
    kd = pl.program_id(1)
    tile = meta_ref[0, g]
    rid = tile * _TM + lax.broadcasted_iota(jnp.int32, (_TM, 1), 0)
    m = (rid >= meta_ref[2, g]) & (rid < meta_ref[3, g])
    xblk = jnp.where(m, xs_ref[...], 0.0)
    h = jnp.maximum(
        jnp.dot(xblk, w1_ref[0], preferred_element_type=jnp.float32), 0.0)
    y = jnp.dot(h, w2_ref[0], preferred_element_type=jnp.float32)

    first = (kd == 0) & ((g == 0) | (tile != meta_ref[0, jnp.maximum(g - 1, 0)]))

    @pl.when(first)
    def _():
        ys_ref[...] = y

    @pl.when(jnp.logical_not(first))
    def _():
        ys_ref[...] += y


def _gmm(xs, w1, w2, meta):
    spec = pltpu.PrefetchScalarGridSpec(
        num_scalar_prefetch=1,
        grid=(_G, _KD),
        in_specs=[
            pl.BlockSpec((_TM, _D), lambda g, kd, m: (m[0, g], 0)),
            pl.BlockSpec((1, _D, _TKD), lambda g, kd, m: (m[1, g], 0, kd)),
            pl.BlockSpec((1, _TKD, _D), lambda g, kd, m: (m[1, g], kd, 0)),
        ],
        out_specs=pl.BlockSpec((_TM, _D), lambda g, kd, m: (m[0, g], 0)),
    )
    return pl.pallas_call(
        _gmm_body,
        grid_spec=spec,
        out_shape=jax.ShapeDtypeStruct((_A, _D), jnp.float32),
    )(meta, xs, w1, w2)


def _combine_gather(ys, dest01):
    tpw = _N // _NW
    ch = 32
    nch = tpw // ch
    mesh = plsc.VectorSubcoreMesh(core_axis_name="c", subcore_axis_name="s")
    sds = jax.ShapeDtypeStruct((_N, _D), jnp.float32)

    @functools.partial(
        pl.kernel,
        out_type=(sds, sds),
        mesh=mesh,
        scratch_types=[
            pltpu.VMEM((ch, _D), jnp.float32),
            pltpu.VMEM((ch, _D), jnp.float32),
            pltpu.VMEM((ch,), jnp.int32),
            pltpu.VMEM((ch,), jnp.int32),
            pltpu.SemaphoreType.DMA,
            pltpu.SemaphoreType.DMA,
        ],
    )
    def k(ys_hbm, d_hbm, y0_hbm, y1_hbm, r0_v, r1_v, i0_v, i1_v, s0, s1):
        wid = lax.axis_index("s") * _NC + lax.axis_index("c")
        base = wid * tpw
        for cnk in range(nch):
            t0 = base + cnk * ch
            pltpu.sync_copy(d_hbm.at[0, pl.ds(t0, ch)], i0_v)
            pltpu.sync_copy(d_hbm.at[1, pl.ds(t0, ch)], i1_v)
            c0 = pltpu.async_copy(ys_hbm.at[i0_v], r0_v, s0)
            c1 = pltpu.async_copy(ys_hbm.at[i1_v], r1_v, s1)
            c0.wait()
            c1.wait()
            pltpu.sync_copy(r0_v, y0_hbm.at[pl.ds(t0, ch)])
            pltpu.sync_copy(r1_v, y1_hbm.at[pl.ds(t0, ch)])

    return k(ys, dest01)


def _combine_body(y0_ref, y1_ref, ew_ref, out_ref):
    ew = ew_ref[...]
    out_ref[...] = y0_ref[...] * ew[:, 0:1] + y1_ref[...] * ew[:, 1:2]


def _combine(y0, y1, ew):
    tme = 512
    return pl.pallas_call(
        _combine_body,
        grid=(_N // tme,),
        in_specs=[
            pl.BlockSpec((tme, _D), lambda i: (i, 0)),
            pl.BlockSpec((tme, _D), lambda i: (i, 0)),
            pl.BlockSpec((tme, _TOPK), lambda i: (i, 0)),
        ],
        out_specs=pl.BlockSpec((tme, _D), lambda i: (i, 0)),
        out_shape=jax.ShapeDtypeStruct((_N, _D), jnp.float32),
    )(y0, y1, ew)


def kernel(x, expert_weights, W1, W2, expert_indices, batch_size_per_expert):
    ei2d = expert_indices.reshape(-1).astype(jnp.int32).reshape(128, 128)
    cnt = batch_size_per_expert.reshape(1, _E).astype(jnp.int32)

    dest2d, meta = _dest(ei2d, cnt)
    dest01 = dest2d.reshape(_N, _TOPK).T

    xs = _dispatch(x, dest01)
    ys = _gmm(xs, W1, W2, meta)
    yu0, yu1 = _combine_gather(ys, dest01)
    return _combine(yu0, yu1, expert_weights)


# speedup vs baseline: 1.0293x; 1.0293x over previous
import functools

import jax
import jax.numpy as jnp
from jax import lax
from jax.experimental import pallas as pl
from jax.experimental.pallas import tpu as pltpu
import jax.experimental.pallas.tpu_sc as plsc

_E = 8
_TOPK = 2
_N = 8192
_D = 1024
_DFF = 4096
_A = _N * _TOPK

_TM = 1024
_TKD = 2048
_NT = _A // _TM
_G = _NT + _E - 1
_KD = _DFF // _TKD

_NC = 2
_NS = 16
_NW = _NC * _NS


def _dest_body(ei_ref, cnt_ref, dest_ref, meta_ref):
    ei = ei_ref[...]

    r = lax.broadcasted_iota(jnp.int32, (128, 128), 0)
    c = lax.broadcasted_iota(jnp.int32, (128, 128), 1)
    tlt = (r < c).astype(jnp.float32)
    slo = (c < r).astype(jnp.float32)

    dest = jnp.zeros((128, 128), jnp.float32)
    offs = [jnp.int32(0)]
    for e in range(_E):
        plane = (ei == e).astype(jnp.float32)
        col_ex = jnp.dot(plane, tlt, preferred_element_type=jnp.float32)
        rowsum = jnp.sum(plane, axis=1, keepdims=True)
        row_ex = jnp.dot(slo, rowsum, preferred_element_type=jnp.float32)
        rank = col_ex + row_ex
        dest = dest + plane * (offs[e].astype(jnp.float32) + rank)
        offs.append(offs[e] + cnt_ref[0, e])
    dest_ref[...] = dest.astype(jnp.int32)

    start_t, units = [], [jnp.int32(0)]
    for e in range(_E):
        start_t.append(offs[e] // _TM)
        cnt_e = offs[e + 1] - offs[e]
        end_t = jnp.where(cnt_e > 0, (offs[e + 1] - 1) // _TM, start_t[e] - 1)
        units.append(units[e] + jnp.maximum(end_t - start_t[e] + 1, 0))
    g = lax.broadcasted_iota(jnp.int32, (1, 128), 1)
    e_id = jnp.zeros((1, 128), jnp.int32)
    for e in range(_E - 1):
        e_id = e_id + (g >= units[e + 1]).astype(jnp.int32)
    e_id = jnp.minimum(e_id, _E - 1)
    tile = jnp.zeros((1, 128), jnp.int32)
    off_e = jnp.zeros((1, 128), jnp.int32)
    end_e = jnp.zeros((1, 128), jnp.int32)
    for e in range(_E):
        sel = (e_id == e).astype(jnp.int32)
        tile = tile + sel * (start_t[e] + g - units[e])
        off_e = off_e + sel * offs[e]
        end_e = end_e + sel * offs[e + 1]
    valid = (g < units[_E]).astype(jnp.int32)
    tile = jnp.where(valid == 1, tile, _NT - 1)
    lo = valid * jnp.maximum(off_e, tile * _TM)
    hi = valid * jnp.minimum(end_e, (tile + 1) * _TM)
    e_row = valid * e_id
    z = jnp.zeros((1, 128), jnp.int32)
    meta_ref[...] = jnp.concatenate(
        [tile, e_row, lo, hi, z, z, z, z], axis=0)


def _dest(ei2d, cnt):
    return pl.pallas_call(
        _dest_body,
        in_specs=[
            pl.BlockSpec((128, 128), lambda: (0, 0)),
            pl.BlockSpec(memory_space=pltpu.SMEM),
        ],
        out_shape=(jax.ShapeDtypeStruct((128, 128), jnp.int32),
                   jax.ShapeDtypeStruct((8, 128), jnp.int32)),
    )(ei2d, cnt)


def _dispatch(x, dest01):
    tpw = _N // _NW
    ch = 64
    nch = tpw // ch
    mesh = plsc.VectorSubcoreMesh(core_axis_name="c", subcore_axis_name="s")

    @functools.partial(
        pl.kernel,
        out_type=jax.ShapeDtypeStruct((_A, _D), jnp.float32),
        mesh=mesh,
        scratch_types=[
            pltpu.VMEM((ch, _D), jnp.float32),
            pltpu.VMEM((ch,), jnp.int32),
            pltpu.VMEM((ch,), jnp.int32),
            pltpu.SemaphoreType.DMA,
            pltpu.SemaphoreType.DMA,
        ],
    )
    def k(x_hbm, d_hbm, xs_hbm, rows_v, i0_v, i1_v, s0, s1):
        wid = lax.axis_index("s") * _NC + lax.axis_index("c")
        base = wid * tpw
        for cnk in range(nch):
            t0 = base + cnk * ch
            pltpu.sync_copy(x_hbm.at[pl.ds(t0, ch)], rows_v)
            pltpu.sync_copy(d_hbm.at[0, pl.ds(t0, ch)], i0_v)
            pltpu.sync_copy(d_hbm.at[1, pl.ds(t0, ch)], i1_v)
            c0 = pltpu.async_copy(rows_v, xs_hbm.at[i0_v], s0)
            c1 = pltpu.async_copy(rows_v, xs_hbm.at[i1_v], s1)
            c0.wait()
            c1.wait()

    return k(x, dest01)


def _gmm_body(meta_ref, xs_ref, w1_ref, w2_ref, ys_ref):
    g = pl.program_id(0)
    kd = pl.program_id(1)
    tile = meta_ref[0, g]
    rid = tile * _TM + lax.broadcasted_iota(jnp.int32, (_TM, 1), 0)
    m = (rid >= meta_ref[2, g]) & (rid < meta_ref[3, g])
    xblk = jnp.where(m, xs_ref[...], 0.0)
    h = jnp.maximum(
        jnp.dot(xblk, w1_ref[0], preferred_element_type=jnp.float32), 0.0)
    y = jnp.dot(h, w2_ref[0], preferred_element_type=jnp.float32)

    first = (kd == 0) & ((g == 0) | (tile != meta_ref[0, jnp.maximum(g - 1, 0)]))

    @pl.when(first)
    def _():
        ys_ref[...] = y

    @pl.when(jnp.logical_not(first))
    def _():
        ys_ref[...] += y


def _gmm(xs, w1, w2, meta):
    spec = pltpu.PrefetchScalarGridSpec(
        num_scalar_prefetch=1,
        grid=(_G, _KD),
        in_specs=[
            pl.BlockSpec((_TM, _D), lambda g, kd, m: (m[0, g], 0)),
            pl.BlockSpec((1, _D, _TKD), lambda g, kd, m: (m[1, g], 0, kd)),
            pl.BlockSpec((1, _TKD, _D), lambda g, kd, m: (m[1, g], kd, 0)),
        ],
        out_specs=pl.BlockSpec((_TM, _D), lambda g, kd, m: (m[0, g], 0)),
    )
    return pl.pallas_call(
        _gmm_body,
        grid_spec=spec,
        out_shape=jax.ShapeDtypeStruct((_A, _D), jnp.float32),
    )(meta, xs, w1, w2)


def _combine_gather(ys, dest01):
    tpw = _N // _NW
    ch = 32
    nch = tpw // ch
    mesh = plsc.VectorSubcoreMesh(core_axis_name="c", subcore_axis_name="s")
    sds = jax.ShapeDtypeStruct((_N, _D), jnp.float32)

    @functools.partial(
        pl.kernel,
        out_type=(sds, sds),
        mesh=mesh,
        scratch_types=[
            pltpu.VMEM((ch, _D), jnp.float32),
            pltpu.VMEM((ch, _D), jnp.float32),
            pltpu.VMEM((ch,), jnp.int32),
            pltpu.VMEM((ch,), jnp.int32),
            pltpu.SemaphoreType.DMA,
            pltpu.SemaphoreType.DMA,
        ],
    )
    def k(ys_hbm, d_hbm, y0_hbm, y1_hbm, r0_v, r1_v, i0_v, i1_v, s0, s1):
        wid = lax.axis_index("s") * _NC + lax.axis_index("c")
        base = wid * tpw
        for cnk in range(nch):
            t0 = base + cnk * ch
            pltpu.sync_copy(d_hbm.at[0, pl.ds(t0, ch)], i0_v)
            pltpu.sync_copy(d_hbm.at[1, pl.ds(t0, ch)], i1_v)
            c0 = pltpu.async_copy(ys_hbm.at[i0_v], r0_v, s0)
            c1 = pltpu.async_copy(ys_hbm.at[i1_v], r1_v, s1)
            c0.wait()
            c1.wait()
            pltpu.sync_copy(r0_v, y0_hbm.at[pl.ds(t0, ch)])
            pltpu.sync_copy(r1_v, y1_hbm.at[pl.ds(t0, ch)])

    return k(ys, dest01)


def _combine_body(y0_ref, y1_ref, ew_ref, out_ref):
    ew = ew_ref[...]
    out_ref[...] = y0_ref[...] * ew[:, 0:1] + y1_ref[...] * ew[:, 1:2]


def _combine(y0, y1, ew):
    tme = 512
    return pl.pallas_call(
        _combine_body,
        grid=(_N // tme,),
        in_specs=[
            pl.BlockSpec((tme, _D), lambda i: (i, 0)),
            pl.BlockSpec((tme, _D), lambda i: (i, 0)),
            pl.BlockSpec((tme, _TOPK), lambda i: (i, 0)),
        ],
        out_specs=pl.BlockSpec((tme, _D), lambda i: (i, 0)),
        out_shape=jax.ShapeDtypeStruct((_N, _D), jnp.float32),
    )(y0, y1, ew)


def kernel(x, expert_weights, W1, W2, expert_indices, batch_size_per_expert):
    ei2d = expert_indices.reshape(-1).astype(jnp.int32).reshape(128, 128)
    cnt = batch_size_per_expert.reshape(1, _E).astype(jnp.int32)

    dest2d, meta = _dest(ei2d, cnt)
    dest01 = dest2d.reshape(_N, _TOPK).T

    xs = _dispatch(x, dest01)
    ys = _gmm(xs, W1, W2, meta)
    yu0, yu1 = _combine_gather(ys, dest01)
    return _combine(yu0, yu1, expert_weights)


# speedup vs baseline: 1.0538x; 1.0238x over previous
import functools

import jax
import jax.numpy as jnp
from jax import lax
from jax.experimental import pallas as pl
from jax.experimental.pallas import tpu as pltpu
import jax.experimental.pallas.tpu_sc as plsc

_E = 8
_TOPK = 2
_N = 8192
_D = 1024
_DFF = 4096
_A = _N * _TOPK

_TM = 1024
_TKD = 2048
_NT = _A // _TM
_G = _NT + _E - 1
_KD = _DFF // _TKD

_NC = 2
_NS = 16
_NW = _NC * _NS


def _dest_body(ei_ref, cnt_ref, dest_ref, meta_ref):
    ei = ei_ref[...]

    r = lax.broadcasted_iota(jnp.int32, (128, 128), 0)
    c = lax.broadcasted_iota(jnp.int32, (128, 128), 1)
    tlt = (r < c).astype(jnp.float32)
    slo = (c < r).astype(jnp.float32)

    dest = jnp.zeros((128, 128), jnp.float32)
    offs = [jnp.int32(0)]
    for e in range(_E):
        plane = (ei == e).astype(jnp.float32)
        col_ex = jnp.dot(plane, tlt, preferred_element_type=jnp.float32)
        rowsum = jnp.sum(plane, axis=1, keepdims=True)
        row_ex = jnp.dot(slo, rowsum, preferred_element_type=jnp.float32)
        rank = col_ex + row_ex
        dest = dest + plane * (offs[e].astype(jnp.float32) + rank)
        offs.append(offs[e] + cnt_ref[0, e])
    dest_ref[...] = dest.astype(jnp.int32)

    start_t, units = [], [jnp.int32(0)]
    for e in range(_E):
        start_t.append(offs[e] // _TM)
        cnt_e = offs[e + 1] - offs[e]
        end_t = jnp.where(cnt_e > 0, (offs[e + 1] - 1) // _TM, start_t[e] - 1)
        units.append(units[e] + jnp.maximum(end_t - start_t[e] + 1, 0))
    g = lax.broadcasted_iota(jnp.int32, (1, 128), 1)
    e_id = jnp.zeros((1, 128), jnp.int32)
    for e in range(_E - 1):
        e_id = e_id + (g >= units[e + 1]).astype(jnp.int32)
    e_id = jnp.minimum(e_id, _E - 1)
    tile = jnp.zeros((1, 128), jnp.int32)
    off_e = jnp.zeros((1, 128), jnp.int32)
    end_e = jnp.zeros((1, 128), jnp.int32)
    for e in range(_E):
        sel = (e_id == e).astype(jnp.int32)
        tile = tile + sel * (start_t[e] + g - units[e])
        off_e = off_e + sel * offs[e]
        end_e = end_e + sel * offs[e + 1]
    valid = (g < units[_E]).astype(jnp.int32)
    tile = jnp.where(valid == 1, tile, _NT - 1)
    lo = valid * jnp.maximum(off_e, tile * _TM)
    hi = valid * jnp.minimum(end_e, (tile + 1) * _TM)
    e_row = valid * e_id
    z = jnp.zeros((1, 128), jnp.int32)
    meta_ref[...] = jnp.concatenate(
        [tile, e_row, lo, hi, z, z, z, z], axis=0)


def _dest(ei2d, cnt):
    return pl.pallas_call(
        _dest_body,
        in_specs=[
            pl.BlockSpec((128, 128), lambda: (0, 0)),
            pl.BlockSpec(memory_space=pltpu.SMEM),
        ],
        out_shape=(jax.ShapeDtypeStruct((128, 128), jnp.int32),
                   jax.ShapeDtypeStruct((8, 128), jnp.int32)),
    )(ei2d, cnt)


def _dispatch(x, dest01):
    tpw = _N // _NW
    ch = 32
    nch = tpw // ch
    mesh = plsc.VectorSubcoreMesh(core_axis_name="c", subcore_axis_name="s")

    @functools.partial(
        pl.kernel,
        out_type=jax.ShapeDtypeStruct((_A, _D), jnp.float32),
        mesh=mesh,
        scratch_types=[
            pltpu.VMEM((2, ch, _D), jnp.float32),
            pltpu.VMEM((2, ch), jnp.int32),
            pltpu.VMEM((2, ch), jnp.int32),
            pltpu.SemaphoreType.DMA((2,)),
            pltpu.SemaphoreType.DMA((2,)),
            pltpu.SemaphoreType.DMA((2,)),
            pltpu.SemaphoreType.DMA((2,)),
            pltpu.SemaphoreType.DMA((2,)),
        ],
    )
    def k(x_hbm, d_hbm, xs_hbm, rows_v, i0_v, i1_v, lr, l0, l1, s0, s1):
        wid = lax.axis_index("s") * _NC + lax.axis_index("c")
        base = wid * tpw

        def loads(c):
            b = c % 2
            t0 = base + c * ch
            return (
                pltpu.async_copy(x_hbm.at[pl.ds(t0, ch)], rows_v.at[b], lr.at[b]),
                pltpu.async_copy(d_hbm.at[0, pl.ds(t0, ch)], i0_v.at[b], l0.at[b]),
                pltpu.async_copy(d_hbm.at[1, pl.ds(t0, ch)], i1_v.at[b], l1.at[b]),
            )

        ld = {0: loads(0)}
        sc = {}
        for c in range(nch):
            b = c % 2
            if c + 1 < nch:
                if c - 1 >= 0:
                    sc[c - 1][0].wait()
                    sc[c - 1][1].wait()
                ld[c + 1] = loads(c + 1)
            for cp in ld[c]:
                cp.wait()
            sc[c] = (
                pltpu.async_copy(rows_v.at[b], xs_hbm.at[i0_v.at[b]], s0.at[b]),
                pltpu.async_copy(rows_v.at[b], xs_hbm.at[i1_v.at[b]], s1.at[b]),
            )
        for c in (nch - 2, nch - 1):
            sc[c][0].wait()
            sc[c][1].wait()

    return k(x, dest01)


def _gmm_body(meta_ref, xs_ref, w1_ref, w2_ref, ys_ref):
    g = pl.program_id(0)
    kd = pl.program_id(1)
    tile = meta_ref[0, g]
    rid = tile * _TM + lax.broadcasted_iota(jnp.int32, (_TM, 1), 0)
    m = (rid >= meta_ref[2, g]) & (rid < meta_ref[3, g])
    xblk = jnp.where(m, xs_ref[...], 0.0)
    h = jnp.maximum(
        jnp.dot(xblk, w1_ref[0], preferred_element_type=jnp.float32), 0.0)
    y = jnp.dot(h, w2_ref[0], preferred_element_type=jnp.float32)

    first = (kd == 0) & ((g == 0) | (tile != meta_ref[0, jnp.maximum(g - 1, 0)]))

    @pl.when(first)
    def _():
        ys_ref[...] = y

    @pl.when(jnp.logical_not(first))
    def _():
        ys_ref[...] += y


def _gmm(xs, w1, w2, meta):
    spec = pltpu.PrefetchScalarGridSpec(
        num_scalar_prefetch=1,
        grid=(_G, _KD),
        in_specs=[
            pl.BlockSpec((_TM, _D), lambda g, kd, m: (m[0, g], 0)),
            pl.BlockSpec((1, _D, _TKD), lambda g, kd, m: (m[1, g], 0, kd)),
            pl.BlockSpec((1, _TKD, _D), lambda g, kd, m: (m[1, g], kd, 0)),
        ],
        out_specs=pl.BlockSpec((_TM, _D), lambda g, kd, m: (m[0, g], 0)),
    )
    return pl.pallas_call(
        _gmm_body,
        grid_spec=spec,
        out_shape=jax.ShapeDtypeStruct((_A, _D), jnp.float32),
    )(meta, xs, w1, w2)


def _combine_gather(ys, dest01):
    tpw = _N // _NW
    ch = 16
    nch = tpw // ch
    mesh = plsc.VectorSubcoreMesh(core_axis_name="c", subcore_axis_name="s")
    sds = jax.ShapeDtypeStruct((_N, _D), jnp.float32)

    @functools.partial(
        pl.kernel,
        out_type=(sds, sds),
        mesh=mesh,
        scratch_types=[
            pltpu.VMEM((2, ch, _D), jnp.float32),
            pltpu.VMEM((2, ch, _D), jnp.float32),
            pltpu.VMEM((2, ch), jnp.int32),
            pltpu.VMEM((2, ch), jnp.int32),
            pltpu.SemaphoreType.DMA((2,)),
            pltpu.SemaphoreType.DMA((2,)),
            pltpu.SemaphoreType.DMA((2,)),
            pltpu.SemaphoreType.DMA((2,)),
            pltpu.SemaphoreType.DMA((2,)),
            pltpu.SemaphoreType.DMA((2,)),
        ],
    )
    def k(ys_hbm, d_hbm, y0_hbm, y1_hbm, r0_v, r1_v, i0_v, i1_v,
          l0, l1, g0, g1, w0, w1):
        wid = lax.axis_index("s") * _NC + lax.axis_index("c")
        base = wid * tpw

        def loads(c):
            b = c % 2
            t0 = base + c * ch
            return (
                pltpu.async_copy(d_hbm.at[0, pl.ds(t0, ch)], i0_v.at[b], l0.at[b]),
                pltpu.async_copy(d_hbm.at[1, pl.ds(t0, ch)], i1_v.at[b], l1.at[b]),
            )

        ld = {0: loads(0)}
        st = {}
        for c in range(nch):
            b = c % 2
            t0 = base + c * ch
            if c + 1 < nch:
                ld[c + 1] = loads(c + 1)
            for cp in ld[c]:
                cp.wait()
            if c - 2 >= 0:
                st[c - 2][0].wait()
                st[c - 2][1].wait()
            ga = (
                pltpu.async_copy(ys_hbm.at[i0_v.at[b]], r0_v.at[b], g0.at[b]),
                pltpu.async_copy(ys_hbm.at[i1_v.at[b]], r1_v.at[b], g1.at[b]),
            )
            ga[0].wait()
            ga[1].wait()
            st[c] = (
                pltpu.async_copy(r0_v.at[b], y0_hbm.at[pl.ds(t0, ch)], w0.at[b]),
                pltpu.async_copy(r1_v.at[b], y1_hbm.at[pl.ds(t0, ch)], w1.at[b]),
            )
        for c in (nch - 2, nch - 1):
            st[c][0].wait()
            st[c][1].wait()

    return k(ys, dest01)


def _combine_body(y0_ref, y1_ref, ew_ref, out_ref):
    ew = ew_ref[...]
    out_ref[...] = y0_ref[...] * ew[:, 0:1] + y1_ref[...] * ew[:, 1:2]


def _combine(y0, y1, ew):
    tme = 512
    return pl.pallas_call(
        _combine_body,
        grid=(_N // tme,),
        in_specs=[
            pl.BlockSpec((tme, _D), lambda i: (i, 0)),
            pl.BlockSpec((tme, _D), lambda i: (i, 0)),
            pl.BlockSpec((tme, _TOPK), lambda i: (i, 0)),
        ],
        out_specs=pl.BlockSpec((tme, _D), lambda i: (i, 0)),
        out_shape=jax.ShapeDtypeStruct((_N, _D), jnp.float32),
    )(y0, y1, ew)


def kernel(x, expert_weights, W1, W2, expert_indices, batch_size_per_expert):
    ei2d = expert_indices.reshape(-1).astype(jnp.int32).reshape(128, 128)
    cnt = batch_size_per_expert.reshape(1, _E).astype(jnp.int32)

    dest2d, meta = _dest(ei2d, cnt)
    dest01 = dest2d.reshape(_N, _TOPK).T

    xs = _dispatch(x, dest01)
    ys = _gmm(xs, W1, W2, meta)
    yu0, yu1 = _combine_gather(ys, dest01)
    return _combine(yu0, yu1, expert_weights)
